# baseline (speedup 1.0000x reference)
import jax
import jax.numpy as jnp
from jax import lax
from jax.experimental import pallas as pl
from jax.experimental.pallas import tpu as pltpu
from jax.experimental.pallas import tpu_sc as plsc

N_NODES = 10000
N_EDGES = 320000
D = 128

NC = 2
NS = 16
NW = NC * NS
CH = 128
N_CHK = 80
E_REAL = N_EDGES // NW
E_IDX = (N_CHK + 2) * CH
N_PAD = 10240
ROWS_T = N_PAD // NS


def _make_sc_agg():
    mesh = plsc.VectorSubcoreMesh(
        core_axis_name="c", subcore_axis_name="s", num_cores=NC, num_subcores=NS
    )
    out_type = jax.ShapeDtypeStruct((NC, N_PAD, D), jnp.float32)
    scratch = [
        pltpu.VMEM((CH,), jnp.int32),
        pltpu.VMEM((CH,), jnp.int32),
        pltpu.VMEM((CH,), jnp.int32),
        pltpu.VMEM((CH,), jnp.int32),
        pltpu.VMEM((CH, D), jnp.float32),
        pltpu.VMEM((CH, D), jnp.float32),
        pltpu.VMEM_SHARED((N_PAD, D), jnp.float32),
        pltpu.SemaphoreType.DMA,
        pltpu.SemaphoreType.DMA,
        pltpu.SemaphoreType.DMA,
        pltpu.SemaphoreType.DMA,
    ]

    def body(h_hbm, src_hbm, dst_hbm, znd_hbm, out_hbm,
             src0, dst0, src1, dst1, rows0, rows1, acc_s,
             gs0, gs1, is0, is1):
        c = lax.axis_index("c")
        s = lax.axis_index("s")
        wid = s * NC + c
        r0 = s * ROWS_T

        for k in range(ROWS_T // CH):
            rr = r0 + k * CH
            pltpu.sync_copy(znd_hbm.at[pl.ds(rr, CH)], rows0)
            pltpu.sync_copy(rows0, acc_s.at[pl.ds(rr, CH)])
        plsc.subcore_barrier()

        base = wid * E_IDX
        bufs = ((src0, dst0, rows0, gs0, is0), (src1, dst1, rows1, gs1, is1))

        def half(a, b, pre_off):
            (sA, dA, rA, gsA, isA) = a
            (sB, dB, rB, gsB, isB) = b
            pltpu.make_async_copy(h_hbm.at[sA], rA, gsA).wait()
            pltpu.make_async_copy(src_hbm.at[pl.ds(0, CH)], sB, isB).wait()
            pltpu.make_async_copy(dst_hbm.at[pl.ds(0, CH)], dB, isB).wait()
            pltpu.async_copy(h_hbm.at[sB], rB, gsB)
            pltpu.sync_copy(rA, acc_s.at[dA], add=True)
            pltpu.async_copy(src_hbm.at[pl.ds(pre_off, CH)], sA, isA)
            pltpu.async_copy(dst_hbm.at[pl.ds(pre_off, CH)], dA, isA)

        pltpu.sync_copy(src_hbm.at[pl.ds(base, CH)], src0)
        pltpu.sync_copy(dst_hbm.at[pl.ds(base, CH)], dst0)
        pltpu.async_copy(h_hbm.at[src0], rows0, gs0)
        pltpu.async_copy(src_hbm.at[pl.ds(base + CH, CH)], src1, is1)
        pltpu.async_copy(dst_hbm.at[pl.ds(base + CH, CH)], dst1, is1)

        def pair(p, carry):
            g = base + 2 * p * CH
            half(bufs[0], bufs[1], g + 2 * CH)
            half(bufs[1], bufs[0], g + 3 * CH)
            return carry

        lax.fori_loop(0, N_CHK // 2, pair, 0)

        pltpu.make_async_copy(h_hbm.at[src0], rows0, gs0).wait()
        pltpu.make_async_copy(src_hbm.at[pl.ds(0, CH)], src1, is1).wait()
        pltpu.make_async_copy(dst_hbm.at[pl.ds(0, CH)], dst1, is1).wait()
        plsc.subcore_barrier()

        for k in range(ROWS_T // CH):
            rr = r0 + k * CH
            pltpu.sync_copy(acc_s.at[pl.ds(rr, CH)], rows0)
            pltpu.sync_copy(rows0, out_hbm.at[c, pl.ds(rr, CH)])

    return pl.kernel(body, out_type, mesh=mesh, scratch_types=scratch)


_sc_agg = _make_sc_agg()

_BR = 1000


def _tc_layer_body(h_ref, aA_ref, aB_ref, dA_ref, dB_ref, ws_ref, wn_ref,
                   b_ref, o_ref):
    deg = jnp.maximum(dA_ref[:, 0:1] + dB_ref[:, 0:1], 1.0)
    hn = (aA_ref[...] + aB_ref[...]) / deg
    acc = jnp.dot(h_ref[...], ws_ref[...], preferred_element_type=jnp.float32)
    acc = acc + jnp.dot(hn, wn_ref[...], preferred_element_type=jnp.float32)
    acc = acc + b_ref[...]
    o_ref[...] = jnp.maximum(acc, 0.01 * acc)


_tc_layer = pl.pallas_call(
    _tc_layer_body,
    grid=(N_NODES // _BR,),
    in_specs=[
        pl.BlockSpec((_BR, D), lambda i: (i, 0)),
        pl.BlockSpec((_BR, D), lambda i: (i, 0)),
        pl.BlockSpec((_BR, D), lambda i: (i, 0)),
        pl.BlockSpec((_BR, D), lambda i: (i, 0)),
        pl.BlockSpec((_BR, D), lambda i: (i, 0)),
        pl.BlockSpec((D, D), lambda i: (0, 0)),
        pl.BlockSpec((D, D), lambda i: (0, 0)),
        pl.BlockSpec((1, D), lambda i: (0, 0)),
    ],
    out_specs=pl.BlockSpec((_BR, D), lambda i: (i, 0)),
    out_shape=jax.ShapeDtypeStruct((N_NODES, D), jnp.float32),
)


def kernel(n_feat, edge_index, Ws0, Wn0, b0, Ws1, Wn1, b1, Ws2, Wn2, b2):
    src = edge_index[0].astype(jnp.int32)
    dst = edge_index[1].astype(jnp.int32)
    srcp = (jnp.zeros((NW, E_IDX), jnp.int32)
            .at[:, :E_REAL].set(src.reshape(NW, E_REAL)).reshape(-1))
    pad_dst = N_NODES + jnp.arange(E_IDX, dtype=jnp.int32) % (N_PAD - N_NODES)
    dstp = (jnp.broadcast_to(pad_dst, (NW, E_IDX))
            .at[:, :E_REAL].set(dst.reshape(NW, E_REAL)).reshape(-1))
    znd = jnp.zeros((N_PAD, D), jnp.float32)
    ones_feat = jnp.ones((N_NODES, D), jnp.float32)

    dgp = _sc_agg(ones_feat, srcp, dstp, znd)
    dA, dB = dgp[0], dgp[1]

    h = n_feat
    agg = _sc_agg(h, srcp, dstp, znd)
    h = _tc_layer(h, agg[0], agg[1], dA, dB, Ws0, Wn0, b0.reshape(1, D))
    agg = _sc_agg(h, srcp, dstp, znd)
    h = _tc_layer(h, agg[0], agg[1], dA, dB, Ws1, Wn1, b1.reshape(1, D))
    agg = _sc_agg(h, srcp, dstp, znd)
    h = _tc_layer(h, agg[0], agg[1], dA, dB, Ws2, Wn2, b2.reshape(1, D))
    return h


# speedup vs baseline: 1.0544x; 1.0544x over previous
import jax
import jax.numpy as jnp
from jax import lax
from jax.experimental import pallas as pl
from jax.experimental.pallas import tpu as pltpu
from jax.experimental.pallas import tpu_sc as plsc

N_NODES = 10000
N_EDGES = 320000
D = 128

NC = 2
NS = 16
NW = NC * NS
CH = 80
N_CHK = 128
E_REAL = N_EDGES // NW
E_IDX = (N_CHK + 2) * CH
N_PAD = 10240
ROWS_T = N_PAD // NS


def _make_sc_agg():
    mesh = plsc.VectorSubcoreMesh(
        core_axis_name="c", subcore_axis_name="s", num_cores=NC, num_subcores=NS
    )
    out_type = jax.ShapeDtypeStruct((NC, N_PAD, D), jnp.float32)
    scratch = [
        pltpu.VMEM((CH,), jnp.int32),
        pltpu.VMEM((CH,), jnp.int32),
        pltpu.VMEM((CH,), jnp.int32),
        pltpu.VMEM((CH,), jnp.int32),
        pltpu.VMEM((CH, D), jnp.float32),
        pltpu.VMEM((CH, D), jnp.float32),
        pltpu.VMEM_SHARED((N_PAD, D), jnp.float32),
        pltpu.SemaphoreType.DMA,
        pltpu.SemaphoreType.DMA,
        pltpu.SemaphoreType.DMA,
        pltpu.SemaphoreType.DMA,
    ]

    def body(h_hbm, src_hbm, dst_hbm, znd_hbm, out_hbm,
             src0, dst0, src1, dst1, rows0, rows1, acc_s,
             gs0, gs1, is0, is1):
        c = lax.axis_index("c")
        s = lax.axis_index("s")
        wid = s * NC + c
        r0 = s * ROWS_T

        for k in range(ROWS_T // CH):
            rr = r0 + k * CH
            pltpu.sync_copy(znd_hbm.at[pl.ds(rr, CH)], rows0)
            pltpu.sync_copy(rows0, acc_s.at[pl.ds(rr, CH)])
        plsc.subcore_barrier()

        base = wid * E_IDX
        bufs = ((src0, dst0, rows0, gs0, is0), (src1, dst1, rows1, gs1, is1))

        def half(a, b, pre_off):
            (sA, dA, rA, gsA, isA) = a
            (sB, dB, rB, gsB, isB) = b
            pltpu.make_async_copy(h_hbm.at[sA], rA, gsA).wait()
            pltpu.make_async_copy(src_hbm.at[pl.ds(0, CH)], sB, isB).wait()
            pltpu.make_async_copy(dst_hbm.at[pl.ds(0, CH)], dB, isB).wait()
            pltpu.async_copy(h_hbm.at[sB], rB, gsB)
            pltpu.sync_copy(rA, acc_s.at[dA], add=True)
            pltpu.async_copy(src_hbm.at[pl.ds(pre_off, CH)], sA, isA)
            pltpu.async_copy(dst_hbm.at[pl.ds(pre_off, CH)], dA, isA)

        pltpu.sync_copy(src_hbm.at[pl.ds(base, CH)], src0)
        pltpu.sync_copy(dst_hbm.at[pl.ds(base, CH)], dst0)
        pltpu.async_copy(h_hbm.at[src0], rows0, gs0)
        pltpu.async_copy(src_hbm.at[pl.ds(base + CH, CH)], src1, is1)
        pltpu.async_copy(dst_hbm.at[pl.ds(base + CH, CH)], dst1, is1)

        def pair(p, carry):
            g = base + 2 * p * CH
            half(bufs[0], bufs[1], g + 2 * CH)
            half(bufs[1], bufs[0], g + 3 * CH)
            return carry

        lax.fori_loop(0, N_CHK // 2, pair, 0)

        pltpu.make_async_copy(h_hbm.at[src0], rows0, gs0).wait()
        pltpu.make_async_copy(src_hbm.at[pl.ds(0, CH)], src1, is1).wait()
        pltpu.make_async_copy(dst_hbm.at[pl.ds(0, CH)], dst1, is1).wait()
        plsc.subcore_barrier()

        for k in range(ROWS_T // CH):
            rr = r0 + k * CH
            pltpu.sync_copy(acc_s.at[pl.ds(rr, CH)], rows0)
            pltpu.sync_copy(rows0, out_hbm.at[c, pl.ds(rr, CH)])

    return pl.kernel(body, out_type, mesh=mesh, scratch_types=scratch)


_sc_agg = _make_sc_agg()

_BR = 1000


def _tc_layer_body(h_ref, aA_ref, aB_ref, dA_ref, dB_ref, ws_ref, wn_ref,
                   b_ref, o_ref):
    deg = jnp.maximum(dA_ref[:, 0:1] + dB_ref[:, 0:1], 1.0)
    hn = (aA_ref[...] + aB_ref[...]) / deg
    acc = jnp.dot(h_ref[...], ws_ref[...], preferred_element_type=jnp.float32)
    acc = acc + jnp.dot(hn, wn_ref[...], preferred_element_type=jnp.float32)
    acc = acc + b_ref[...]
    o_ref[...] = jnp.maximum(acc, 0.01 * acc)


_tc_layer = pl.pallas_call(
    _tc_layer_body,
    grid=(N_NODES // _BR,),
    in_specs=[
        pl.BlockSpec((_BR, D), lambda i: (i, 0)),
        pl.BlockSpec((_BR, D), lambda i: (i, 0)),
        pl.BlockSpec((_BR, D), lambda i: (i, 0)),
        pl.BlockSpec((_BR, D), lambda i: (i, 0)),
        pl.BlockSpec((_BR, D), lambda i: (i, 0)),
        pl.BlockSpec((D, D), lambda i: (0, 0)),
        pl.BlockSpec((D, D), lambda i: (0, 0)),
        pl.BlockSpec((1, D), lambda i: (0, 0)),
    ],
    out_specs=pl.BlockSpec((_BR, D), lambda i: (i, 0)),
    out_shape=jax.ShapeDtypeStruct((N_NODES, D), jnp.float32),
)


def kernel(n_feat, edge_index, Ws0, Wn0, b0, Ws1, Wn1, b1, Ws2, Wn2, b2):
    src = edge_index[0].astype(jnp.int32)
    dst = edge_index[1].astype(jnp.int32)
    srcp = (jnp.zeros((NW, E_IDX), jnp.int32)
            .at[:, :E_REAL].set(src.reshape(NW, E_REAL)).reshape(-1))
    pad_dst = N_NODES + jnp.arange(E_IDX, dtype=jnp.int32) % (N_PAD - N_NODES)
    dstp = (jnp.broadcast_to(pad_dst, (NW, E_IDX))
            .at[:, :E_REAL].set(dst.reshape(NW, E_REAL)).reshape(-1))
    znd = jnp.zeros((N_PAD, D), jnp.float32)
    ones_feat = jnp.ones((N_NODES, D), jnp.float32)

    dgp = _sc_agg(ones_feat, srcp, dstp, znd)
    dA, dB = dgp[0], dgp[1]

    h = n_feat
    agg = _sc_agg(h, srcp, dstp, znd)
    h = _tc_layer(h, agg[0], agg[1], dA, dB, Ws0, Wn0, b0.reshape(1, D))
    agg = _sc_agg(h, srcp, dstp, znd)
    h = _tc_layer(h, agg[0], agg[1], dA, dB, Ws1, Wn1, b1.reshape(1, D))
    agg = _sc_agg(h, srcp, dstp, znd)
    h = _tc_layer(h, agg[0], agg[1], dA, dB, Ws2, Wn2, b2.reshape(1, D))
    return h


# speedup vs baseline: 1.0832x; 1.0273x over previous
import jax
import jax.numpy as jnp
from jax import lax
from jax.experimental import pallas as pl
from jax.experimental.pallas import tpu as pltpu
from jax.experimental.pallas import tpu_sc as plsc

N_NODES = 10000
N_EDGES = 320000
D = 128

NC = 2
NS = 16
NW = NC * NS
CH = 128
N_CHK = 80
E_REAL = N_EDGES // NW
E_IDX = N_CHK * CH
N_PAD = 10240
ROWS_T = N_PAD // NS


def _make_sc_agg():
    mesh = plsc.VectorSubcoreMesh(
        core_axis_name="c", subcore_axis_name="s", num_cores=NC, num_subcores=NS
    )
    out_type = jax.ShapeDtypeStruct((NC, N_PAD, D), jnp.float32)
    scratch = [
        pltpu.VMEM((CH,), jnp.int32),
        pltpu.VMEM((CH,), jnp.int32),
        pltpu.VMEM((CH, D), jnp.float32),
        pltpu.VMEM_SHARED((N_PAD, D), jnp.float32),
        pltpu.SemaphoreType.DMA,
    ]

    def body(h_hbm, src_hbm, dst_hbm, znd_hbm, out_hbm,
             src_v, dst_v, rows_v, acc_s, sem):
        c = lax.axis_index("c")
        s = lax.axis_index("s")
        wid = s * NC + c
        r0 = s * ROWS_T

        for k in range(ROWS_T // CH):
            rr = r0 + k * CH
            pltpu.sync_copy(znd_hbm.at[pl.ds(rr, CH)], rows_v)
            pltpu.sync_copy(rows_v, acc_s.at[pl.ds(rr, CH)])
        plsc.subcore_barrier()

        base = wid * E_IDX

        def step(i, carry):
            off = base + i * CH
            pltpu.sync_copy(src_hbm.at[pl.ds(off, CH)], src_v)
            pltpu.sync_copy(dst_hbm.at[pl.ds(off, CH)], dst_v)
            pltpu.async_copy(h_hbm.at[src_v], rows_v, sem).wait()
            pltpu.sync_copy(rows_v, acc_s.at[dst_v], add=True)
            return carry

        lax.fori_loop(0, N_CHK, step, 0)
        plsc.subcore_barrier()

        for k in range(ROWS_T // CH):
            rr = r0 + k * CH
            pltpu.sync_copy(acc_s.at[pl.ds(rr, CH)], rows_v)
            pltpu.sync_copy(rows_v, out_hbm.at[c, pl.ds(rr, CH)])

    return pl.kernel(body, out_type, mesh=mesh, scratch_types=scratch)


_sc_agg = _make_sc_agg()

_BR = 1000


def _tc_layer_body(h_ref, aA_ref, aB_ref, dA_ref, dB_ref, ws_ref, wn_ref,
                   b_ref, o_ref):
    deg = jnp.maximum(dA_ref[:, 0:1] + dB_ref[:, 0:1], 1.0)
    hn = (aA_ref[...] + aB_ref[...]) / deg
    acc = jnp.dot(h_ref[...], ws_ref[...], preferred_element_type=jnp.float32)
    acc = acc + jnp.dot(hn, wn_ref[...], preferred_element_type=jnp.float32)
    acc = acc + b_ref[...]
    o_ref[...] = jnp.maximum(acc, 0.01 * acc)


_tc_layer = pl.pallas_call(
    _tc_layer_body,
    grid=(N_NODES // _BR,),
    in_specs=[
        pl.BlockSpec((_BR, D), lambda i: (i, 0)),
        pl.BlockSpec((_BR, D), lambda i: (i, 0)),
        pl.BlockSpec((_BR, D), lambda i: (i, 0)),
        pl.BlockSpec((_BR, D), lambda i: (i, 0)),
        pl.BlockSpec((_BR, D), lambda i: (i, 0)),
        pl.BlockSpec((D, D), lambda i: (0, 0)),
        pl.BlockSpec((D, D), lambda i: (0, 0)),
        pl.BlockSpec((1, D), lambda i: (0, 0)),
    ],
    out_specs=pl.BlockSpec((_BR, D), lambda i: (i, 0)),
    out_shape=jax.ShapeDtypeStruct((N_NODES, D), jnp.float32),
)


def kernel(n_feat, edge_index, Ws0, Wn0, b0, Ws1, Wn1, b1, Ws2, Wn2, b2):
    src = edge_index[0].astype(jnp.int32)
    dst = edge_index[1].astype(jnp.int32)
    srcp = (jnp.zeros((NW, E_IDX), jnp.int32)
            .at[:, :E_REAL].set(src.reshape(NW, E_REAL)).reshape(-1))
    pad_dst = N_NODES + jnp.arange(E_IDX, dtype=jnp.int32) % (N_PAD - N_NODES)
    dstp = (jnp.broadcast_to(pad_dst, (NW, E_IDX))
            .at[:, :E_REAL].set(dst.reshape(NW, E_REAL)).reshape(-1))
    znd = jnp.zeros((N_PAD, D), jnp.float32)
    ones_feat = jnp.ones((N_NODES, D), jnp.float32)

    dgp = _sc_agg(ones_feat, srcp, dstp, znd)
    dA, dB = dgp[0], dgp[1]

    h = n_feat
    agg = _sc_agg(h, srcp, dstp, znd)
    h = _tc_layer(h, agg[0], agg[1], dA, dB, Ws0, Wn0, b0.reshape(1, D))
    agg = _sc_agg(h, srcp, dstp, znd)
    h = _tc_layer(h, agg[0], agg[1], dA, dB, Ws1, Wn1, b1.reshape(1, D))
    agg = _sc_agg(h, srcp, dstp, znd)
    h = _tc_layer(h, agg[0], agg[1], dA, dB, Ws2, Wn2, b2.reshape(1, D))
    return h


# speedup vs baseline: 1.1911x; 1.0996x over previous
import jax
import jax.numpy as jnp
from jax import lax
from jax.experimental import pallas as pl
from jax.experimental.pallas import tpu as pltpu
from jax.experimental.pallas import tpu_sc as plsc

N_NODES = 10000
N_EDGES = 320000
D = 128

NC = 2
NS = 16
NW = NC * NS
CH = 80
N_CHK = 128
E_REAL = N_EDGES // NW
E_IDX = N_CHK * CH
N_PAD = 10240
ROWS_T = N_PAD // NS


def _make_sc_agg():
    mesh = plsc.VectorSubcoreMesh(
        core_axis_name="c", subcore_axis_name="s", num_cores=NC, num_subcores=NS
    )
    out_type = jax.ShapeDtypeStruct((NC, N_PAD, D), jnp.float32)
    scratch = [
        pltpu.VMEM((CH,), jnp.int32),
        pltpu.VMEM((CH,), jnp.int32),
        pltpu.VMEM((CH, D), jnp.float32),
        pltpu.VMEM_SHARED((N_PAD, D), jnp.float32),
        pltpu.SemaphoreType.DMA,
    ]

    def body(h_hbm, src_hbm, dst_hbm, znd_hbm, out_hbm,
             src_v, dst_v, rows_v, acc_s, sem):
        c = lax.axis_index("c")
        s = lax.axis_index("s")
        wid = s * NC + c
        r0 = s * ROWS_T

        for k in range(ROWS_T // CH):
            rr = r0 + k * CH
            pltpu.sync_copy(znd_hbm.at[pl.ds(rr, CH)], rows_v)
            pltpu.sync_copy(rows_v, acc_s.at[pl.ds(rr, CH)])
        plsc.subcore_barrier()

        base = wid * E_IDX

        def step(i, carry):
            off = base + i * CH
            pltpu.sync_copy(src_hbm.at[pl.ds(off, CH)], src_v)
            pltpu.sync_copy(dst_hbm.at[pl.ds(off, CH)], dst_v)
            pltpu.async_copy(h_hbm.at[src_v], rows_v, sem).wait()
            pltpu.sync_copy(rows_v, acc_s.at[dst_v], add=True)
            return carry

        lax.fori_loop(0, N_CHK, step, 0)
        plsc.subcore_barrier()

        for k in range(ROWS_T // CH):
            rr = r0 + k * CH
            pltpu.sync_copy(acc_s.at[pl.ds(rr, CH)], rows_v)
            pltpu.sync_copy(rows_v, out_hbm.at[c, pl.ds(rr, CH)])

    return pl.kernel(body, out_type, mesh=mesh, scratch_types=scratch)


_sc_agg = _make_sc_agg()


def _make_sc_deg():
    mesh = plsc.VectorSubcoreMesh(
        core_axis_name="c", subcore_axis_name="s", num_cores=NC, num_subcores=NS
    )
    out_type = jax.ShapeDtypeStruct((NC, N_PAD, D), jnp.float32)
    scratch = [
        pltpu.VMEM((CH,), jnp.int32),
        pltpu.VMEM((CH, D), jnp.float32),
        pltpu.VMEM_SHARED((N_PAD, D), jnp.float32),
    ]

    def body(ones_hbm, dst_hbm, znd_hbm, out_hbm, dst_v, rows_v, acc_s):
        c = lax.axis_index("c")
        s = lax.axis_index("s")
        wid = s * NC + c
        r0 = s * ROWS_T

        for k in range(ROWS_T // CH):
            rr = r0 + k * CH
            pltpu.sync_copy(znd_hbm.at[pl.ds(rr, CH)], rows_v)
            pltpu.sync_copy(rows_v, acc_s.at[pl.ds(rr, CH)])
        plsc.subcore_barrier()

        pltpu.sync_copy(ones_hbm, rows_v)
        base = wid * E_IDX

        def step(i, carry):
            off = base + i * CH
            pltpu.sync_copy(dst_hbm.at[pl.ds(off, CH)], dst_v)
            pltpu.sync_copy(rows_v, acc_s.at[dst_v], add=True)
            return carry

        lax.fori_loop(0, N_CHK, step, 0)
        plsc.subcore_barrier()

        for k in range(ROWS_T // CH):
            rr = r0 + k * CH
            pltpu.sync_copy(acc_s.at[pl.ds(rr, CH)], rows_v)
            pltpu.sync_copy(rows_v, out_hbm.at[c, pl.ds(rr, CH)])

    return pl.kernel(body, out_type, mesh=mesh, scratch_types=scratch)


_sc_deg = _make_sc_deg()

_BR = 1000


def _tc_layer_body(h_ref, aA_ref, aB_ref, dA_ref, dB_ref, ws_ref, wn_ref,
                   b_ref, o_ref):
    deg = jnp.maximum(dA_ref[:, 0:1] + dB_ref[:, 0:1], 1.0)
    hn = (aA_ref[...] + aB_ref[...]) / deg
    acc = jnp.dot(h_ref[...], ws_ref[...], preferred_element_type=jnp.float32)
    acc = acc + jnp.dot(hn, wn_ref[...], preferred_element_type=jnp.float32)
    acc = acc + b_ref[...]
    o_ref[...] = jnp.maximum(acc, 0.01 * acc)


_tc_layer = pl.pallas_call(
    _tc_layer_body,
    grid=(N_NODES // _BR,),
    in_specs=[
        pl.BlockSpec((_BR, D), lambda i: (i, 0)),
        pl.BlockSpec((_BR, D), lambda i: (i, 0)),
        pl.BlockSpec((_BR, D), lambda i: (i, 0)),
        pl.BlockSpec((_BR, D), lambda i: (i, 0)),
        pl.BlockSpec((_BR, D), lambda i: (i, 0)),
        pl.BlockSpec((D, D), lambda i: (0, 0)),
        pl.BlockSpec((D, D), lambda i: (0, 0)),
        pl.BlockSpec((1, D), lambda i: (0, 0)),
    ],
    out_specs=pl.BlockSpec((_BR, D), lambda i: (i, 0)),
    out_shape=jax.ShapeDtypeStruct((N_NODES, D), jnp.float32),
)


def kernel(n_feat, edge_index, Ws0, Wn0, b0, Ws1, Wn1, b1, Ws2, Wn2, b2):
    src = edge_index[0].astype(jnp.int32)
    dst = edge_index[1].astype(jnp.int32)
    srcp = (jnp.zeros((NW, E_IDX), jnp.int32)
            .at[:, :E_REAL].set(src.reshape(NW, E_REAL)).reshape(-1))
    pad_dst = N_NODES + jnp.arange(E_IDX, dtype=jnp.int32) % (N_PAD - N_NODES)
    dstp = (jnp.broadcast_to(pad_dst, (NW, E_IDX))
            .at[:, :E_REAL].set(dst.reshape(NW, E_REAL)).reshape(-1))
    znd = jnp.zeros((N_PAD, D), jnp.float32)
    ones_rows = jnp.ones((CH, D), jnp.float32)

    dgp = _sc_deg(ones_rows, dstp, znd)
    dA, dB = dgp[0], dgp[1]

    h = n_feat
    agg = _sc_agg(h, srcp, dstp, znd)
    h = _tc_layer(h, agg[0], agg[1], dA, dB, Ws0, Wn0, b0.reshape(1, D))
    agg = _sc_agg(h, srcp, dstp, znd)
    h = _tc_layer(h, agg[0], agg[1], dA, dB, Ws1, Wn1, b1.reshape(1, D))
    agg = _sc_agg(h, srcp, dstp, znd)
    h = _tc_layer(h, agg[0], agg[1], dA, dB, Ws2, Wn2, b2.reshape(1, D))
    return h


# speedup vs baseline: 2.0371x; 1.7102x over previous
import jax
import jax.numpy as jnp
from jax import lax
from jax.experimental import pallas as pl
from jax.experimental.pallas import tpu as pltpu
from jax.experimental.pallas import tpu_sc as plsc

N_NODES = 10000
N_EDGES = 320000
D = 128

NC = 2
NS = 16
NW = NC * NS
CH = 80
E_REAL = N_EDGES // NW
N_CHK = E_REAL // CH
E_IDX = E_REAL
N_PAD = 10240
ROWS_T = N_PAD // NS


def _make_sc_agg():
    mesh = plsc.VectorSubcoreMesh(
        core_axis_name="c", subcore_axis_name="s", num_cores=NC, num_subcores=NS
    )
    out_type = jax.ShapeDtypeStruct((NC, N_PAD, D), jnp.float32)
    scratch = [
        pltpu.VMEM((CH,), jnp.int32),
        pltpu.VMEM((CH,), jnp.int32),
        pltpu.VMEM((CH, D), jnp.float32),
        pltpu.VMEM_SHARED((N_PAD, D), jnp.float32),
        pltpu.SemaphoreType.DMA,
    ]

    def body(h_hbm, src_hbm, dst_hbm, znd_hbm, out_hbm,
             src_v, dst_v, rows_v, acc_s, sem):
        c = lax.axis_index("c")
        s = lax.axis_index("s")
        wid = s * NC + c
        r0 = s * ROWS_T

        for k in range(ROWS_T // CH):
            rr = r0 + k * CH
            pltpu.sync_copy(znd_hbm.at[pl.ds(rr, CH)], rows_v)
            pltpu.sync_copy(rows_v, acc_s.at[pl.ds(rr, CH)])
        plsc.subcore_barrier()

        base = wid * E_IDX

        def step(i, carry):
            off = base + i * CH
            pltpu.sync_copy(src_hbm.at[pl.ds(off, CH)], src_v)
            pltpu.sync_copy(dst_hbm.at[pl.ds(off, CH)], dst_v)
            pltpu.async_copy(h_hbm.at[src_v], rows_v, sem).wait()
            pltpu.sync_copy(rows_v, acc_s.at[dst_v], add=True)
            return carry

        lax.fori_loop(0, N_CHK, step, 0)
        plsc.subcore_barrier()

        for k in range(ROWS_T // CH):
            rr = r0 + k * CH
            pltpu.sync_copy(acc_s.at[pl.ds(rr, CH)], rows_v)
            pltpu.sync_copy(rows_v, out_hbm.at[c, pl.ds(rr, CH)])

    return pl.kernel(body, out_type, mesh=mesh, scratch_types=scratch)


_sc_agg = _make_sc_agg()


def _make_sc_deg():
    mesh = plsc.VectorSubcoreMesh(
        core_axis_name="c", subcore_axis_name="s", num_cores=NC, num_subcores=NS
    )
    out_type = jax.ShapeDtypeStruct((NC, N_PAD, D), jnp.float32)
    scratch = [
        pltpu.VMEM((CH,), jnp.int32),
        pltpu.VMEM((CH, D), jnp.float32),
        pltpu.VMEM_SHARED((N_PAD, D), jnp.float32),
    ]

    def body(ones_hbm, dst_hbm, znd_hbm, out_hbm, dst_v, rows_v, acc_s):
        c = lax.axis_index("c")
        s = lax.axis_index("s")
        wid = s * NC + c
        r0 = s * ROWS_T

        for k in range(ROWS_T // CH):
            rr = r0 + k * CH
            pltpu.sync_copy(znd_hbm.at[pl.ds(rr, CH)], rows_v)
            pltpu.sync_copy(rows_v, acc_s.at[pl.ds(rr, CH)])
        plsc.subcore_barrier()

        pltpu.sync_copy(ones_hbm, rows_v)
        base = wid * E_IDX

        def step(i, carry):
            off = base + i * CH
            pltpu.sync_copy(dst_hbm.at[pl.ds(off, CH)], dst_v)
            pltpu.sync_copy(rows_v, acc_s.at[dst_v], add=True)
            return carry

        lax.fori_loop(0, N_CHK, step, 0)
        plsc.subcore_barrier()

        for k in range(ROWS_T // CH):
            rr = r0 + k * CH
            pltpu.sync_copy(acc_s.at[pl.ds(rr, CH)], rows_v)
            pltpu.sync_copy(rows_v, out_hbm.at[c, pl.ds(rr, CH)])

    return pl.kernel(body, out_type, mesh=mesh, scratch_types=scratch)


_sc_deg = _make_sc_deg()

_BR = 1000


def _tc_layer_body(h_ref, aA_ref, aB_ref, dA_ref, dB_ref, ws_ref, wn_ref,
                   b_ref, o_ref):
    deg = jnp.maximum(dA_ref[:, 0:1] + dB_ref[:, 0:1], 1.0)
    hn = (aA_ref[...] + aB_ref[...]) / deg
    acc = jnp.dot(h_ref[...], ws_ref[...], preferred_element_type=jnp.float32)
    acc = acc + jnp.dot(hn, wn_ref[...], preferred_element_type=jnp.float32)
    acc = acc + b_ref[...]
    o_ref[...] = jnp.maximum(acc, 0.01 * acc)


_tc_layer = pl.pallas_call(
    _tc_layer_body,
    grid=(N_NODES // _BR,),
    in_specs=[
        pl.BlockSpec((_BR, D), lambda i: (i, 0)),
        pl.BlockSpec((_BR, D), lambda i: (i, 0)),
        pl.BlockSpec((_BR, D), lambda i: (i, 0)),
        pl.BlockSpec((_BR, D), lambda i: (i, 0)),
        pl.BlockSpec((_BR, D), lambda i: (i, 0)),
        pl.BlockSpec((D, D), lambda i: (0, 0)),
        pl.BlockSpec((D, D), lambda i: (0, 0)),
        pl.BlockSpec((1, D), lambda i: (0, 0)),
    ],
    out_specs=pl.BlockSpec((_BR, D), lambda i: (i, 0)),
    out_shape=jax.ShapeDtypeStruct((N_NODES, D), jnp.float32),
)


def kernel(n_feat, edge_index, Ws0, Wn0, b0, Ws1, Wn1, b1, Ws2, Wn2, b2):
    srcp = edge_index[0].astype(jnp.int32)
    dstp = edge_index[1].astype(jnp.int32)
    znd = jnp.zeros((N_PAD, D), jnp.float32)
    ones_rows = jnp.ones((CH, D), jnp.float32)

    dgp = _sc_deg(ones_rows, dstp, znd)
    dA, dB = dgp[0], dgp[1]

    h = n_feat
    agg = _sc_agg(h, srcp, dstp, znd)
    h = _tc_layer(h, agg[0], agg[1], dA, dB, Ws0, Wn0, b0.reshape(1, D))
    agg = _sc_agg(h, srcp, dstp, znd)
    h = _tc_layer(h, agg[0], agg[1], dA, dB, Ws1, Wn1, b1.reshape(1, D))
    agg = _sc_agg(h, srcp, dstp, znd)
    h = _tc_layer(h, agg[0], agg[1], dA, dB, Ws2, Wn2, b2.reshape(1, D))
    return h


# speedup vs baseline: 2.3325x; 1.1450x over previous
import jax
import jax.numpy as jnp
from jax import lax
from jax.experimental import pallas as pl
from jax.experimental.pallas import tpu as pltpu
from jax.experimental.pallas import tpu_sc as plsc

N_NODES = 10000
N_EDGES = 320000
D = 128

NC = 2
NS = 16
NW = NC * NS
CH = 80
E_REAL = N_EDGES // NW
N_CHK = E_REAL // CH
E_IDX = E_REAL
N_PAD = 10240
ROWS_T = N_PAD // NS


def _make_sc_agg():
    mesh = plsc.VectorSubcoreMesh(
        core_axis_name="c", subcore_axis_name="s", num_cores=NC, num_subcores=NS
    )
    out_type = jax.ShapeDtypeStruct((NC, N_PAD, D), jnp.float32)
    scratch = [
        pltpu.VMEM((CH,), jnp.int32),
        pltpu.VMEM((CH,), jnp.int32),
        pltpu.VMEM((CH, D), jnp.float32),
        pltpu.VMEM_SHARED((N_PAD, D), jnp.float32),
        pltpu.SemaphoreType.DMA,
    ]

    def body(h_hbm, src_hbm, dst_hbm, znd_hbm, out_hbm,
             src_v, dst_v, rows_v, acc_s, sem):
        c = lax.axis_index("c")
        s = lax.axis_index("s")
        wid = s * NC + c
        r0 = s * ROWS_T

        for k in range(ROWS_T // CH):
            rr = r0 + k * CH
            pltpu.sync_copy(znd_hbm.at[pl.ds(rr, CH)], rows_v)
            pltpu.sync_copy(rows_v, acc_s.at[pl.ds(rr, CH)])
        plsc.subcore_barrier()

        base = wid * E_IDX

        def step(i, carry):
            off = base + i * CH
            ca = pltpu.async_copy(src_hbm.at[pl.ds(off, CH)], src_v, sem)
            cb = pltpu.async_copy(dst_hbm.at[pl.ds(off, CH)], dst_v, sem)
            ca.wait()
            cb.wait()
            pltpu.async_copy(h_hbm.at[src_v], rows_v, sem).wait()
            pltpu.sync_copy(rows_v, acc_s.at[dst_v], add=True)
            return carry

        lax.fori_loop(0, N_CHK, step, 0)
        plsc.subcore_barrier()

        for k in range(ROWS_T // CH):
            rr = r0 + k * CH
            pltpu.sync_copy(acc_s.at[pl.ds(rr, CH)], rows_v)
            pltpu.sync_copy(rows_v, out_hbm.at[c, pl.ds(rr, CH)])

    return pl.kernel(body, out_type, mesh=mesh, scratch_types=scratch)


_sc_agg = _make_sc_agg()


def _make_sc_deg():
    mesh = plsc.VectorSubcoreMesh(
        core_axis_name="c", subcore_axis_name="s", num_cores=NC, num_subcores=NS
    )
    out_type = jax.ShapeDtypeStruct((NC, N_PAD, D), jnp.float32)
    scratch = [
        pltpu.VMEM((CH,), jnp.int32),
        pltpu.VMEM((CH, D), jnp.float32),
        pltpu.VMEM_SHARED((N_PAD, D), jnp.float32),
    ]

    def body(ones_hbm, dst_hbm, znd_hbm, out_hbm, dst_v, rows_v, acc_s):
        c = lax.axis_index("c")
        s = lax.axis_index("s")
        wid = s * NC + c
        r0 = s * ROWS_T

        for k in range(ROWS_T // CH):
            rr = r0 + k * CH
            pltpu.sync_copy(znd_hbm.at[pl.ds(rr, CH)], rows_v)
            pltpu.sync_copy(rows_v, acc_s.at[pl.ds(rr, CH)])
        plsc.subcore_barrier()

        pltpu.sync_copy(ones_hbm, rows_v)
        base = wid * E_IDX

        def step(i, carry):
            off = base + i * CH
            pltpu.sync_copy(dst_hbm.at[pl.ds(off, CH)], dst_v)
            pltpu.sync_copy(rows_v, acc_s.at[dst_v], add=True)
            return carry

        lax.fori_loop(0, N_CHK, step, 0)
        plsc.subcore_barrier()

        for k in range(ROWS_T // CH):
            rr = r0 + k * CH
            pltpu.sync_copy(acc_s.at[pl.ds(rr, CH)], rows_v)
            pltpu.sync_copy(rows_v, out_hbm.at[c, pl.ds(rr, CH)])

    return pl.kernel(body, out_type, mesh=mesh, scratch_types=scratch)


_sc_deg = _make_sc_deg()

_BR = 1000


def _tc_layer_body(h_ref, aA_ref, aB_ref, dA_ref, dB_ref, ws_ref, wn_ref,
                   b_ref, o_ref):
    deg = jnp.maximum(dA_ref[:, 0:1] + dB_ref[:, 0:1], 1.0)
    hn = (aA_ref[...] + aB_ref[...]) / deg
    acc = jnp.dot(h_ref[...], ws_ref[...], preferred_element_type=jnp.float32)
    acc = acc + jnp.dot(hn, wn_ref[...], preferred_element_type=jnp.float32)
    acc = acc + b_ref[...]
    o_ref[...] = jnp.maximum(acc, 0.01 * acc)


_tc_layer = pl.pallas_call(
    _tc_layer_body,
    grid=(N_NODES // _BR,),
    in_specs=[
        pl.BlockSpec((_BR, D), lambda i: (i, 0)),
        pl.BlockSpec((_BR, D), lambda i: (i, 0)),
        pl.BlockSpec((_BR, D), lambda i: (i, 0)),
        pl.BlockSpec((_BR, D), lambda i: (i, 0)),
        pl.BlockSpec((_BR, D), lambda i: (i, 0)),
        pl.BlockSpec((D, D), lambda i: (0, 0)),
        pl.BlockSpec((D, D), lambda i: (0, 0)),
        pl.BlockSpec((1, D), lambda i: (0, 0)),
    ],
    out_specs=pl.BlockSpec((_BR, D), lambda i: (i, 0)),
    out_shape=jax.ShapeDtypeStruct((N_NODES, D), jnp.float32),
)


def kernel(n_feat, edge_index, Ws0, Wn0, b0, Ws1, Wn1, b1, Ws2, Wn2, b2):
    srcp = edge_index[0].astype(jnp.int32)
    dstp = edge_index[1].astype(jnp.int32)
    znd = jnp.zeros((N_PAD, D), jnp.float32)
    ones_rows = jnp.ones((CH, D), jnp.float32)

    dgp = _sc_deg(ones_rows, dstp, znd)
    dA, dB = dgp[0], dgp[1]

    h = n_feat
    agg = _sc_agg(h, srcp, dstp, znd)
    h = _tc_layer(h, agg[0], agg[1], dA, dB, Ws0, Wn0, b0.reshape(1, D))
    agg = _sc_agg(h, srcp, dstp, znd)
    h = _tc_layer(h, agg[0], agg[1], dA, dB, Ws1, Wn1, b1.reshape(1, D))
    agg = _sc_agg(h, srcp, dstp, znd)
    h = _tc_layer(h, agg[0], agg[1], dA, dB, Ws2, Wn2, b2.reshape(1, D))
    return h


# speedup vs baseline: 2.7562x; 1.1817x over previous
import jax
import jax.numpy as jnp
from jax import lax
from jax.experimental import pallas as pl
from jax.experimental.pallas import tpu as pltpu
from jax.experimental.pallas import tpu_sc as plsc

N_NODES = 10000
N_EDGES = 320000
D = 128

NC = 2
NS = 16
NW = NC * NS
CH = 80
E_REAL = N_EDGES // NW
N_CHK = E_REAL // CH
E_IDX = E_REAL
N_PAD = 10240
ROWS_T = N_PAD // NS


def _make_sc_agg():
    mesh = plsc.VectorSubcoreMesh(
        core_axis_name="c", subcore_axis_name="s", num_cores=NC, num_subcores=NS
    )
    out_type = jax.ShapeDtypeStruct((NC, N_PAD, D), jnp.float32)
    scratch = [
        pltpu.VMEM((CH,), jnp.int32),
        pltpu.VMEM((CH,), jnp.int32),
        pltpu.VMEM((CH,), jnp.int32),
        pltpu.VMEM((CH,), jnp.int32),
        pltpu.VMEM((CH, D), jnp.float32),
        pltpu.VMEM_SHARED((N_PAD, D), jnp.float32),
        pltpu.SemaphoreType.DMA,
        pltpu.SemaphoreType.DMA,
        pltpu.SemaphoreType.DMA,
    ]

    def body(h_hbm, src_hbm, dst_hbm, znd_hbm, out_hbm,
             src0, dst0, src1, dst1, rows_v, acc_s, sem, is0, is1):
        c = lax.axis_index("c")
        s = lax.axis_index("s")
        wid = s * NC + c
        r0 = s * ROWS_T

        for k in range(ROWS_T // CH):
            rr = r0 + k * CH
            pltpu.sync_copy(znd_hbm.at[pl.ds(rr, CH)], rows_v)
            pltpu.sync_copy(rows_v, acc_s.at[pl.ds(rr, CH)])
        plsc.subcore_barrier()

        base = wid * E_IDX

        def idx_load(off, sv, dv, isem):
            pltpu.async_copy(src_hbm.at[pl.ds(off, CH)], sv, isem)
            pltpu.async_copy(dst_hbm.at[pl.ds(off, CH)], dv, isem)

        def idx_wait(sv, dv, isem):
            pltpu.make_async_copy(src_hbm.at[pl.ds(0, CH)], sv, isem).wait()
            pltpu.make_async_copy(dst_hbm.at[pl.ds(0, CH)], dv, isem).wait()

        def gat_scat(sv, dv):
            pltpu.async_copy(h_hbm.at[sv], rows_v, sem).wait()
            pltpu.sync_copy(rows_v, acc_s.at[dv], add=True)

        idx_load(base, src0, dst0, is0)
        idx_wait(src0, dst0, is0)
        idx_load(base + CH, src1, dst1, is1)
        gat_scat(src0, dst0)

        def pair(p, carry):
            off1 = base + CH + 2 * p * CH
            idx_wait(src1, dst1, is1)
            idx_load(off1 + CH, src0, dst0, is0)
            gat_scat(src1, dst1)
            idx_wait(src0, dst0, is0)
            idx_load(off1 + 2 * CH, src1, dst1, is1)
            gat_scat(src0, dst0)
            return carry

        lax.fori_loop(0, (N_CHK - 1) // 2, pair, 0)
        idx_wait(src1, dst1, is1)
        plsc.subcore_barrier()

        for k in range(ROWS_T // CH):
            rr = r0 + k * CH
            pltpu.sync_copy(acc_s.at[pl.ds(rr, CH)], rows_v)
            pltpu.sync_copy(rows_v, out_hbm.at[c, pl.ds(rr, CH)])

    return pl.kernel(body, out_type, mesh=mesh, scratch_types=scratch)


_sc_agg = _make_sc_agg()


def _make_sc_deg():
    mesh = plsc.VectorSubcoreMesh(
        core_axis_name="c", subcore_axis_name="s", num_cores=NC, num_subcores=NS
    )
    out_type = jax.ShapeDtypeStruct((NC, N_PAD, D), jnp.float32)
    scratch = [
        pltpu.VMEM((CH,), jnp.int32),
        pltpu.VMEM((CH, D), jnp.float32),
        pltpu.VMEM_SHARED((N_PAD, D), jnp.float32),
    ]

    def body(ones_hbm, dst_hbm, znd_hbm, out_hbm, dst_v, rows_v, acc_s):
        c = lax.axis_index("c")
        s = lax.axis_index("s")
        wid = s * NC + c
        r0 = s * ROWS_T

        for k in range(ROWS_T // CH):
            rr = r0 + k * CH
            pltpu.sync_copy(znd_hbm.at[pl.ds(rr, CH)], rows_v)
            pltpu.sync_copy(rows_v, acc_s.at[pl.ds(rr, CH)])
        plsc.subcore_barrier()

        pltpu.sync_copy(ones_hbm, rows_v)
        base = wid * E_IDX

        def step(i, carry):
            off = base + i * CH
            pltpu.sync_copy(dst_hbm.at[pl.ds(off, CH)], dst_v)
            pltpu.sync_copy(rows_v, acc_s.at[dst_v], add=True)
            return carry

        lax.fori_loop(0, N_CHK, step, 0)
        plsc.subcore_barrier()

        for k in range(ROWS_T // CH):
            rr = r0 + k * CH
            pltpu.sync_copy(acc_s.at[pl.ds(rr, CH)], rows_v)
            pltpu.sync_copy(rows_v, out_hbm.at[c, pl.ds(rr, CH)])

    return pl.kernel(body, out_type, mesh=mesh, scratch_types=scratch)


_sc_deg = _make_sc_deg()

_BR = 1000


def _tc_layer_body(h_ref, aA_ref, aB_ref, dA_ref, dB_ref, ws_ref, wn_ref,
                   b_ref, o_ref):
    deg = jnp.maximum(dA_ref[:, 0:1] + dB_ref[:, 0:1], 1.0)
    hn = (aA_ref[...] + aB_ref[...]) / deg
    acc = jnp.dot(h_ref[...], ws_ref[...], preferred_element_type=jnp.float32)
    acc = acc + jnp.dot(hn, wn_ref[...], preferred_element_type=jnp.float32)
    acc = acc + b_ref[...]
    o_ref[...] = jnp.maximum(acc, 0.01 * acc)


_tc_layer = pl.pallas_call(
    _tc_layer_body,
    grid=(N_NODES // _BR,),
    in_specs=[
        pl.BlockSpec((_BR, D), lambda i: (i, 0)),
        pl.BlockSpec((_BR, D), lambda i: (i, 0)),
        pl.BlockSpec((_BR, D), lambda i: (i, 0)),
        pl.BlockSpec((_BR, D), lambda i: (i, 0)),
        pl.BlockSpec((_BR, D), lambda i: (i, 0)),
        pl.BlockSpec((D, D), lambda i: (0, 0)),
        pl.BlockSpec((D, D), lambda i: (0, 0)),
        pl.BlockSpec((1, D), lambda i: (0, 0)),
    ],
    out_specs=pl.BlockSpec((_BR, D), lambda i: (i, 0)),
    out_shape=jax.ShapeDtypeStruct((N_NODES, D), jnp.float32),
)


def kernel(n_feat, edge_index, Ws0, Wn0, b0, Ws1, Wn1, b1, Ws2, Wn2, b2):
    pad = jnp.zeros((2 * CH,), jnp.int32)
    srcp = jnp.concatenate([edge_index[0].astype(jnp.int32), pad])
    dstp = jnp.concatenate([edge_index[1].astype(jnp.int32), pad])
    znd = jnp.zeros((N_PAD, D), jnp.float32)
    ones_rows = jnp.ones((CH, D), jnp.float32)

    dgp = _sc_deg(ones_rows, dstp, znd)
    dA, dB = dgp[0], dgp[1]

    h = n_feat
    agg = _sc_agg(h, srcp, dstp, znd)
    h = _tc_layer(h, agg[0], agg[1], dA, dB, Ws0, Wn0, b0.reshape(1, D))
    agg = _sc_agg(h, srcp, dstp, znd)
    h = _tc_layer(h, agg[0], agg[1], dA, dB, Ws1, Wn1, b1.reshape(1, D))
    agg = _sc_agg(h, srcp, dstp, znd)
    h = _tc_layer(h, agg[0], agg[1], dA, dB, Ws2, Wn2, b2.reshape(1, D))
    return h


# speedup vs baseline: 3.0236x; 1.0970x over previous
import jax
import jax.numpy as jnp
from jax import lax
from jax.experimental import pallas as pl
from jax.experimental.pallas import tpu as pltpu
from jax.experimental.pallas import tpu_sc as plsc

N_NODES = 10000
N_EDGES = 320000
D = 128

NC = 2
NS = 16
NW = NC * NS
CH = 80
E_REAL = N_EDGES // NW
N_CHK = E_REAL // CH
E_IDX = E_REAL
N_PAD = 10240
ROWS_T = N_PAD // NS


def _make_sc_agg():
    mesh = plsc.VectorSubcoreMesh(
        core_axis_name="c", subcore_axis_name="s", num_cores=NC, num_subcores=NS
    )
    out_type = jax.ShapeDtypeStruct((NC, N_PAD, D), jnp.float32)
    scratch = [
        pltpu.VMEM((CH,), jnp.int32),
        pltpu.VMEM((CH,), jnp.int32),
        pltpu.VMEM((CH,), jnp.int32),
        pltpu.VMEM((CH,), jnp.int32),
        pltpu.VMEM((CH, D), jnp.float32),
        pltpu.VMEM_SHARED((N_PAD, D), jnp.float32),
        pltpu.SemaphoreType.DMA,
        pltpu.SemaphoreType.DMA,
        pltpu.SemaphoreType.DMA,
    ]

    def body(h_hbm, src_hbm, dst_hbm, znd_hbm, out_hbm,
             src0, dst0, src1, dst1, rows_v, acc_s, sem, is0, is1):
        c = lax.axis_index("c")
        s = lax.axis_index("s")
        wid = s * NC + c
        r0 = s * ROWS_T

        for k in range(ROWS_T // CH):
            rr = r0 + k * CH
            pltpu.sync_copy(znd_hbm.at[pl.ds(rr, CH)], rows_v)
            pltpu.sync_copy(rows_v, acc_s.at[pl.ds(rr, CH)])
        plsc.subcore_barrier()

        base = wid * E_IDX

        def idx_load(off, sv, dv, isem):
            pltpu.async_copy(src_hbm.at[pl.ds(off, CH)], sv, isem)
            pltpu.async_copy(dst_hbm.at[pl.ds(off, CH)], dv, isem)

        def idx_wait(sv, dv, isem):
            pltpu.make_async_copy(src_hbm.at[pl.ds(0, CH)], sv, isem).wait()
            pltpu.make_async_copy(dst_hbm.at[pl.ds(0, CH)], dv, isem).wait()

        def gat_scat(sv, dv):
            pltpu.async_copy(h_hbm.at[sv], rows_v, sem).wait()
            pltpu.sync_copy(rows_v, acc_s.at[dv], add=True)

        idx_load(base, src0, dst0, is0)
        idx_wait(src0, dst0, is0)
        idx_load(base + CH, src1, dst1, is1)
        gat_scat(src0, dst0)

        def pair(p, carry):
            off1 = base + CH + 2 * p * CH
            idx_wait(src1, dst1, is1)
            idx_load(off1 + CH, src0, dst0, is0)
            gat_scat(src1, dst1)
            idx_wait(src0, dst0, is0)
            idx_load(off1 + 2 * CH, src1, dst1, is1)
            gat_scat(src0, dst0)
            return carry

        lax.fori_loop(0, (N_CHK - 1) // 2, pair, 0)
        idx_wait(src1, dst1, is1)
        plsc.subcore_barrier()

        for k in range(ROWS_T // CH):
            rr = r0 + k * CH
            pltpu.sync_copy(acc_s.at[pl.ds(rr, CH)], rows_v)
            pltpu.sync_copy(rows_v, out_hbm.at[c, pl.ds(rr, CH)])

    return pl.kernel(body, out_type, mesh=mesh, scratch_types=scratch)


_sc_agg = _make_sc_agg()


def _make_sc_deg():
    mesh = plsc.VectorSubcoreMesh(
        core_axis_name="c", subcore_axis_name="s", num_cores=NC, num_subcores=NS
    )
    out_type = jax.ShapeDtypeStruct((NC, N_PAD, D), jnp.float32)
    scratch = [
        pltpu.VMEM((CH,), jnp.int32),
        pltpu.VMEM((CH,), jnp.int32),
        pltpu.VMEM((CH, D), jnp.float32),
        pltpu.VMEM_SHARED((N_PAD, D), jnp.float32),
        pltpu.SemaphoreType.DMA,
        pltpu.SemaphoreType.DMA,
    ]

    def body(ones_hbm, dst_hbm, znd_hbm, out_hbm, dst0, dst1, rows_v, acc_s,
             is0, is1):
        c = lax.axis_index("c")
        s = lax.axis_index("s")
        wid = s * NC + c
        r0 = s * ROWS_T

        for k in range(ROWS_T // CH):
            rr = r0 + k * CH
            pltpu.sync_copy(znd_hbm.at[pl.ds(rr, CH)], rows_v)
            pltpu.sync_copy(rows_v, acc_s.at[pl.ds(rr, CH)])
        plsc.subcore_barrier()

        pltpu.sync_copy(ones_hbm, rows_v)
        base = wid * E_IDX

        def idx_wait(dv, isem):
            pltpu.make_async_copy(dst_hbm.at[pl.ds(0, CH)], dv, isem).wait()

        pltpu.async_copy(dst_hbm.at[pl.ds(base, CH)], dst0, is0)
        idx_wait(dst0, is0)
        pltpu.async_copy(dst_hbm.at[pl.ds(base + CH, CH)], dst1, is1)
        pltpu.sync_copy(rows_v, acc_s.at[dst0], add=True)

        def pair(p, carry):
            off1 = base + CH + 2 * p * CH
            idx_wait(dst1, is1)
            pltpu.async_copy(dst_hbm.at[pl.ds(off1 + CH, CH)], dst0, is0)
            pltpu.sync_copy(rows_v, acc_s.at[dst1], add=True)
            idx_wait(dst0, is0)
            pltpu.async_copy(dst_hbm.at[pl.ds(off1 + 2 * CH, CH)], dst1, is1)
            pltpu.sync_copy(rows_v, acc_s.at[dst0], add=True)
            return carry

        lax.fori_loop(0, (N_CHK - 1) // 2, pair, 0)
        idx_wait(dst1, is1)
        plsc.subcore_barrier()

        for k in range(ROWS_T // CH):
            rr = r0 + k * CH
            pltpu.sync_copy(acc_s.at[pl.ds(rr, CH)], rows_v)
            pltpu.sync_copy(rows_v, out_hbm.at[c, pl.ds(rr, CH)])

    return pl.kernel(body, out_type, mesh=mesh, scratch_types=scratch)


_sc_deg = _make_sc_deg()

_BR = 1000


def _tc_layer_body(h_ref, aA_ref, aB_ref, dA_ref, dB_ref, ws_ref, wn_ref,
                   b_ref, o_ref):
    deg = jnp.maximum(dA_ref[0, :, 0:1] + dB_ref[0, :, 0:1], 1.0)
    hn = (aA_ref[0] + aB_ref[0]) / deg
    acc = jnp.dot(h_ref[...], ws_ref[...], preferred_element_type=jnp.float32)
    acc = acc + jnp.dot(hn, wn_ref[...], preferred_element_type=jnp.float32)
    acc = acc + b_ref[...]
    o_ref[...] = jnp.maximum(acc, 0.01 * acc)


_tc_layer = pl.pallas_call(
    _tc_layer_body,
    grid=(N_NODES // _BR,),
    in_specs=[
        pl.BlockSpec((_BR, D), lambda i: (i, 0)),
        pl.BlockSpec((1, _BR, D), lambda i: (0, i, 0)),
        pl.BlockSpec((1, _BR, D), lambda i: (1, i, 0)),
        pl.BlockSpec((1, _BR, D), lambda i: (0, i, 0)),
        pl.BlockSpec((1, _BR, D), lambda i: (1, i, 0)),
        pl.BlockSpec((D, D), lambda i: (0, 0)),
        pl.BlockSpec((D, D), lambda i: (0, 0)),
        pl.BlockSpec((1, D), lambda i: (0, 0)),
    ],
    out_specs=pl.BlockSpec((_BR, D), lambda i: (i, 0)),
    out_shape=jax.ShapeDtypeStruct((N_NODES, D), jnp.float32),
)


def kernel(n_feat, edge_index, Ws0, Wn0, b0, Ws1, Wn1, b1, Ws2, Wn2, b2):
    pad = jnp.zeros((2 * CH,), jnp.int32)
    srcp = jnp.concatenate([edge_index[0].astype(jnp.int32), pad])
    dstp = jnp.concatenate([edge_index[1].astype(jnp.int32), pad])
    znd = jnp.zeros((N_PAD, D), jnp.float32)
    ones_rows = jnp.ones((CH, D), jnp.float32)

    dgp = _sc_deg(ones_rows, dstp, znd)

    h = n_feat
    agg = _sc_agg(h, srcp, dstp, znd)
    h = _tc_layer(h, agg, agg, dgp, dgp, Ws0, Wn0, b0.reshape(1, D))
    agg = _sc_agg(h, srcp, dstp, znd)
    h = _tc_layer(h, agg, agg, dgp, dgp, Ws1, Wn1, b1.reshape(1, D))
    agg = _sc_agg(h, srcp, dstp, znd)
    h = _tc_layer(h, agg, agg, dgp, dgp, Ws2, Wn2, b2.reshape(1, D))
    return h
